# baseline (speedup 1.0000x reference)
import functools

import jax
import jax.numpy as jnp
from jax import lax
from jax.experimental import pallas as pl
from jax.experimental.pallas import tpu as pltpu
from jax.experimental.pallas import tpu_sc as plsc

N = 10000
NP = 10240
E = 320000
NTILE = 16
CHUNK = 128
NCHUNK = 160
GRP = 32
EPT = NCHUNK * CHUNK
EP = EPT * NTILE
RPT = NP // NTILE
B = 128
F32 = jnp.float32


def _sc_mesh():
    return plsc.VectorSubcoreMesh(core_axis_name="c", subcore_axis_name="s")


def _deg_body(dst_hbm, u_hbm, out_hbm, unit_v, idx_v, hist_sh, sem):
    c = lax.axis_index("c")
    s = lax.axis_index("s")
    pltpu.sync_copy(u_hbm.at[pl.ds(0, CHUNK)], unit_v)
    for r in range(RPT // CHUNK):
        pltpu.sync_copy(unit_v, hist_sh.at[pl.ds(s * RPT + r * CHUNK, CHUNK)])
    plsc.subcore_barrier()
    pltpu.sync_copy(u_hbm.at[pl.ds(CHUNK, CHUNK)], unit_v)
    pltpu.sync_copy(dst_hbm.at[c, s], idx_v)

    def chunk(i, carry):
        pltpu.sync_copy(unit_v, hist_sh.at[idx_v.at[i]], add=True)
        return carry

    lax.fori_loop(0, NCHUNK, chunk, 0)
    plsc.subcore_barrier()
    pltpu.sync_copy(hist_sh.at[pl.ds(s * RPT, RPT)],
                    out_hbm.at[pl.ds(c * NP + s * RPT, RPT)])


def _degree_count(dst_st, unit_rows):
    kern = pl.kernel(
        _deg_body,
        out_type=jax.ShapeDtypeStruct((2 * NP, 16), F32),
        mesh=_sc_mesh(),
        scratch_types=[
            pltpu.VMEM((CHUNK, 16), F32),
            pltpu.VMEM((NCHUNK, CHUNK), jnp.int32),
            pltpu.VMEM_SHARED((NP, 16), F32),
            pltpu.SemaphoreType.DMA,
        ],
    )
    return kern(dst_st, unit_rows)


def _agg_body(y_hbm, src_hbm, dst_hbm, z_hbm, out_hbm,
              idxs_v, idxd_v, rows0, rows1, acc_sh, sem0, sem1):
    c = lax.axis_index("c")
    s = lax.axis_index("s")
    pltpu.sync_copy(z_hbm, rows0)
    for r in range(RPT // CHUNK):
        pltpu.sync_copy(rows0, acc_sh.at[pl.ds(s * RPT + r * CHUNK, CHUNK)])
    plsc.subcore_barrier()
    def group(g, carry):
        pltpu.sync_copy(src_hbm.at[c, s, pl.ds(g * GRP, GRP)], idxs_v)
        pltpu.sync_copy(dst_hbm.at[c, s, pl.ds(g * GRP, GRP)], idxd_v)
        pltpu.async_copy(y_hbm.at[idxs_v.at[0]], rows0, sem0)
        pltpu.async_copy(y_hbm.at[idxs_v.at[1]], rows1, sem1)

        def pair(i, carry2):
            i0 = 2 * i
            nxt = jnp.minimum(i0 + 2, GRP - 1)
            nxt1 = jnp.minimum(i0 + 3, GRP - 1)
            pltpu.make_async_copy(y_hbm.at[idxs_v.at[0]], rows0, sem0).wait()
            pltpu.sync_copy(rows0, acc_sh.at[idxd_v.at[i0]], add=True)
            pltpu.async_copy(y_hbm.at[idxs_v.at[nxt]], rows0, sem0)
            pltpu.make_async_copy(y_hbm.at[idxs_v.at[0]], rows1, sem1).wait()
            pltpu.sync_copy(rows1, acc_sh.at[idxd_v.at[i0 + 1]], add=True)
            pltpu.async_copy(y_hbm.at[idxs_v.at[nxt1]], rows1, sem1)
            return carry2

        lax.fori_loop(0, GRP // 2, pair, 0)
        pltpu.make_async_copy(y_hbm.at[idxs_v.at[0]], rows0, sem0).wait()
        pltpu.make_async_copy(y_hbm.at[idxs_v.at[0]], rows1, sem1).wait()
        return carry

    lax.fori_loop(0, NCHUNK // GRP, group, 0)
    plsc.subcore_barrier()
    pltpu.sync_copy(acc_sh.at[pl.ds(s * RPT, RPT)],
                    out_hbm.at[pl.ds(c * NP + s * RPT, RPT)])


def _edge_aggregate(ys, src_st, dst_st, zero_rows, feat):
    kern = pl.kernel(
        _agg_body,
        out_type=jax.ShapeDtypeStruct((2 * NP, feat), F32),
        mesh=_sc_mesh(),
        scratch_types=[
            pltpu.VMEM((GRP, CHUNK), jnp.int32),
            pltpu.VMEM((GRP, CHUNK), jnp.int32),
            pltpu.VMEM((CHUNK, feat), F32),
            pltpu.VMEM((CHUNK, feat), F32),
            pltpu.VMEM_SHARED((NP, feat), F32),
            pltpu.SemaphoreType.DMA,
            pltpu.SemaphoreType.DMA,
        ],
    )
    return kern(ys, src_st, dst_st, zero_rows)


def _onehot_body(b_ref, m_ref):
    rows = lax.broadcasted_iota(jnp.int32, (B, NP), 0)
    m_ref[...] = jnp.where(rows == b_ref[...], 1.0, 0.0).astype(F32)


def _onehot(batch_pad):
    return pl.pallas_call(
        _onehot_body,
        out_shape=jax.ShapeDtypeStruct((B, NP), F32),
    )(batch_pad.reshape(1, NP))


def _y0_body(x_ref, w_ref, cnt_ref, y_ref):
    dis = lax.rsqrt(cnt_ref[:, :1] + 1.0)
    y_ref[...] = jnp.dot(x_ref[...], w_ref[...],
                         preferred_element_type=F32) * dis


def _y0(xs, W1, cnt):
    return pl.pallas_call(
        _y0_body,
        out_shape=jax.ShapeDtypeStruct((2 * NP, W1.shape[1]), F32),
    )(xs, W1, cnt)


def _level_body(agg_ref, y_ref, cnt_ref, b_ref, w1_ref, b1_ref, w2_ref, b2_ref,
                m_ref, wn_ref, pooled_ref, ynext_ref):
    dis = lax.rsqrt(cnt_ref[:, :1] + 1.0)
    h = jax.nn.relu((agg_ref[...] + y_ref[...]) * dis + b_ref[...])
    hid = jax.nn.relu(jnp.dot(h, w1_ref[...], preferred_element_type=F32)
                      + b1_ref[...])
    a = jnp.tanh(jnp.dot(hid, w2_ref[...], preferred_element_type=F32)
                 + b2_ref[...])
    xatt = a * h + h
    m = m_ref[...]
    cntg = jnp.sum(m, axis=1, keepdims=True)
    mean = jnp.dot(m, xatt, preferred_element_type=F32) / jnp.maximum(cntg, 1.0)
    tg = jnp.tanh(mean)
    tgb = lax.dot_general(m, tg, (((0,), (0,)), ((), ())),
                          preferred_element_type=F32)
    coefs = jax.nn.sigmoid(jnp.sum(xatt * tgb, axis=1, keepdims=True))
    pooled_ref[...] = jnp.dot(m, coefs * xatt, preferred_element_type=F32)
    if ynext_ref is not None:
        yn = jnp.dot(h, wn_ref[...], preferred_element_type=F32) * dis
        fn = yn.shape[1]
        if fn < 128:
            yn = jnp.concatenate([yn, jnp.zeros((NP, 128 - fn), F32)], axis=1)
        ynext_ref[...] = yn


def _level(agg, y, cnt, b, w1, b1, w2, b2, m, wn):
    feat = y.shape[1]
    if wn is None:
        out_shape = [jax.ShapeDtypeStruct((B, feat), F32)]
        body = lambda *rs: _level_body(*rs[:9], None, rs[9], None)
        args = (agg, y, cnt, b, w1, b1, w2, b2, m)
    else:
        out_shape = [jax.ShapeDtypeStruct((B, feat), F32),
                     jax.ShapeDtypeStruct((NP, 128), F32)]
        body = _level_body
        args = (agg, y, cnt, b, w1, b1, w2, b2, m, wn)
    return pl.pallas_call(body, out_shape=out_shape)(*args)


def _head_body(p11, p12, p21, p22, p31, p32, w1, t1, c1, w2, t2, c2, w3, t3, c3,
               sw1, sb1, sw2, sb2, fw, fb, out_ref):
    def tnet(e1, e2, w_ref, wbt_ref, bias_ref, d):
        outer = (e1[:, :, None] * e2[:, None, :]).reshape(B, d * d)
        s = jnp.dot(outer, w_ref[...], preferred_element_type=F32)
        comb = jnp.concatenate([e1, e2], axis=1)
        return jax.nn.relu(s + jnp.dot(comb, wbt_ref[...],
                                       preferred_element_type=F32)
                           + bias_ref[...])

    s1 = tnet(p11[...], p12[...], w1, t1, c1, 128)
    s2 = tnet(p21[...], p22[...], w2, t2, c2, 64)
    s3 = tnet(p31[...], p32[...], w3, t3, c3, 32)
    scores = jnp.concatenate([s3, s2, s1], axis=1)
    se = jax.nn.sigmoid(
        jnp.dot(jax.nn.relu(jnp.dot(scores, sw1[...],
                                    preferred_element_type=F32) + sb1[...]),
                sw2[...], preferred_element_type=F32) + sb2[...])
    out_ref[...] = jax.nn.relu(
        jnp.dot(se * scores + scores, fw[...], preferred_element_type=F32)
        + fb[...])


def _head(pools, tn, se_w1, se_b1, se_w2, se_b2, fc_w, fc_b):
    args = list(pools)
    for w2d, wbt, bias in tn:
        args += [w2d, wbt, bias]
    args += [se_w1, se_b1.reshape(1, -1), se_w2, se_b2.reshape(1, -1),
             fc_w, fc_b.reshape(1, -1)]
    return pl.pallas_call(
        _head_body,
        out_shape=jax.ShapeDtypeStruct((B, 64), F32),
    )(*args)


def kernel(x1, x2, edge_index1, edge_index2, batch1, batch2, W1, b1, W2, b2,
           W3, b3, a1_w1, a1_b1, a1_w2, a1_b2, a2_w1, a2_b1, a2_w2, a2_b2,
           a3_w1, a3_b1, a3_w2, a3_b2, tn1_W, tn1_Wb, tn1_bias,
           tn2_W, tn2_Wb, tn2_bias, tn3_W, tn3_Wb, tn3_bias,
           se_w1, se_b1, se_w2, se_b2, fc_w, fc_b):
    pad_e = EP - E
    pad_n = NP - N

    def pad_edges(v, off):
        return jnp.concatenate([v + off, jnp.full((pad_e,), off + NP - 1,
                                                  jnp.int32)])

    src_st = jnp.concatenate([pad_edges(edge_index1[0], 0),
                              pad_edges(edge_index2[0], NP)]
                             ).reshape(2, NTILE, NCHUNK, CHUNK)
    dst_st = jnp.concatenate([pad_edges(edge_index1[1], 0),
                              pad_edges(edge_index2[1], 0)]
                             ).reshape(2, NTILE, NCHUNK, CHUNK)
    xs = jnp.concatenate([
        jnp.pad(x1, ((0, pad_n), (0, 0))),
        jnp.pad(x2, ((0, pad_n), (0, 0)))])
    bp1 = jnp.pad(batch1, (0, pad_n), constant_values=jnp.int32(B + 7))
    bp2 = jnp.pad(batch2, (0, pad_n), constant_values=jnp.int32(B + 7))

    unit_rows = jnp.concatenate([jnp.zeros((CHUNK, 16), F32),
                                 jnp.ones((CHUNK, 16), F32)])

    cnt = _degree_count(dst_st, unit_rows)
    m1 = _onehot(bp1)
    m2 = _onehot(bp2)

    ys = _y0(xs, W1, cnt)
    pools = []
    att = [(a1_w1, a1_b1, a1_w2, a1_b2), (a2_w1, a2_b1, a2_w2, a2_b2),
           (a3_w1, a3_b1, a3_w2, a3_b2)]
    biases = [b1, b2, b3]
    nexts = [W2, W3, None]
    zero_rows = jnp.zeros((CHUNK, 128), F32)
    for lvl in range(3):
        feat = biases[lvl].shape[0]
        agg = _edge_aggregate(ys, src_st, dst_st, zero_rows, 128)
        w1a, b1a, w2a, b2a = att[lvl]
        bl = biases[lvl].reshape(1, feat)
        outs = []
        for g, m in ((0, m1), (1, m2)):
            res = _level(agg[g * NP:(g + 1) * NP, :feat],
                         ys[g * NP:(g + 1) * NP, :feat],
                         cnt[g * NP:(g + 1) * NP], bl,
                         w1a, b1a.reshape(1, -1), w2a, b2a.reshape(1, -1),
                         m, nexts[lvl])
            outs.append(res)
        pools.append((outs[0][0], outs[1][0]))
        if nexts[lvl] is not None:
            ys = jnp.concatenate([outs[0][1], outs[1][1]])

    tn = [(tn1_W.reshape(128 * 128, 64), tn1_Wb.T, tn1_bias.reshape(1, -1)),
          (tn2_W.reshape(64 * 64, 32), tn2_Wb.T, tn2_bias.reshape(1, -1)),
          (tn3_W.reshape(32 * 32, 16), tn3_Wb.T, tn3_bias.reshape(1, -1))]
    pool_args = [pools[0][0], pools[0][1], pools[1][0], pools[1][1],
                 pools[2][0], pools[2][1]]
    return _head(pool_args, tn, se_w1, se_b1, se_w2, se_b2, fc_w, fc_b)


# speedup vs baseline: 1.3299x; 1.3299x over previous
import functools

import jax
import jax.numpy as jnp
from jax import lax
from jax.experimental import pallas as pl
from jax.experimental.pallas import tpu as pltpu
from jax.experimental.pallas import tpu_sc as plsc

N = 10000
NP = 10240
E = 320000
NTILE = 16
CHUNK = 128
NCHUNK = 160
GRP = 32
EPT = NCHUNK * CHUNK
EP = EPT * NTILE
RPT = NP // NTILE
B = 128
F32 = jnp.float32


def _sc_mesh():
    return plsc.VectorSubcoreMesh(core_axis_name="c", subcore_axis_name="s")


def _deg_body(dst_hbm, u_hbm, out_hbm, unit_v, idx_v, hist_sh, sem):
    c = lax.axis_index("c")
    s = lax.axis_index("s")
    pltpu.sync_copy(u_hbm.at[pl.ds(0, CHUNK)], unit_v)
    for r in range(RPT // CHUNK):
        pltpu.sync_copy(unit_v, hist_sh.at[pl.ds(s * RPT + r * CHUNK, CHUNK)])
    plsc.subcore_barrier()
    pltpu.sync_copy(u_hbm.at[pl.ds(CHUNK, CHUNK)], unit_v)
    pltpu.sync_copy(dst_hbm.at[c, s], idx_v)

    def chunk(i, carry):
        pltpu.sync_copy(unit_v, hist_sh.at[idx_v.at[i]], add=True)
        return carry

    lax.fori_loop(0, NCHUNK, chunk, 0)
    plsc.subcore_barrier()
    pltpu.sync_copy(hist_sh.at[pl.ds(s * RPT, RPT)],
                    out_hbm.at[pl.ds(c * NP + s * RPT, RPT)])


def _degree_count(dst_st, unit_rows):
    kern = pl.kernel(
        _deg_body,
        out_type=jax.ShapeDtypeStruct((2 * NP, 16), F32),
        mesh=_sc_mesh(),
        scratch_types=[
            pltpu.VMEM((CHUNK, 16), F32),
            pltpu.VMEM((NCHUNK, CHUNK), jnp.int32),
            pltpu.VMEM_SHARED((NP, 16), F32),
            pltpu.SemaphoreType.DMA,
        ],
    )
    return kern(dst_st, unit_rows)


def _agg_body(y_hbm, src_hbm, dst_hbm, z_hbm, out_hbm,
              idxs_v, idxd_v, rows0, rows1, acc_sh, sem0, sem1):
    c = lax.axis_index("c")
    s = lax.axis_index("s")
    pltpu.sync_copy(z_hbm, rows0)
    for r in range(RPT // CHUNK):
        pltpu.sync_copy(rows0, acc_sh.at[pl.ds(s * RPT + r * CHUNK, CHUNK)])
    plsc.subcore_barrier()
    def group(g, carry):
        pltpu.sync_copy(src_hbm.at[c, s, pl.ds(g * GRP, GRP)], idxs_v)
        pltpu.sync_copy(dst_hbm.at[c, s, pl.ds(g * GRP, GRP)], idxd_v)
        pltpu.async_copy(y_hbm.at[idxs_v.at[0]], rows0, sem0)
        pltpu.async_copy(y_hbm.at[idxs_v.at[1]], rows1, sem1)

        def pair(i, carry2):
            i0 = 2 * i
            nxt = jnp.minimum(i0 + 2, GRP - 1)
            nxt1 = jnp.minimum(i0 + 3, GRP - 1)
            pltpu.make_async_copy(y_hbm.at[idxs_v.at[0]], rows0, sem0).wait()
            pltpu.sync_copy(rows0, acc_sh.at[idxd_v.at[i0]], add=True)
            pltpu.async_copy(y_hbm.at[idxs_v.at[nxt]], rows0, sem0)
            pltpu.make_async_copy(y_hbm.at[idxs_v.at[0]], rows1, sem1).wait()
            pltpu.sync_copy(rows1, acc_sh.at[idxd_v.at[i0 + 1]], add=True)
            pltpu.async_copy(y_hbm.at[idxs_v.at[nxt1]], rows1, sem1)
            return carry2

        lax.fori_loop(0, GRP // 2, pair, 0)
        pltpu.make_async_copy(y_hbm.at[idxs_v.at[0]], rows0, sem0).wait()
        pltpu.make_async_copy(y_hbm.at[idxs_v.at[0]], rows1, sem1).wait()
        return carry

    lax.fori_loop(0, NCHUNK // GRP, group, 0)
    plsc.subcore_barrier()
    pltpu.sync_copy(acc_sh.at[pl.ds(s * RPT, RPT)],
                    out_hbm.at[pl.ds(c * NP + s * RPT, RPT)])


def _agg_narrow_body(y_hbm, src_hbm, dst_hbm, z_hbm, out_hbm,
                     idxs_v, idxd_v, rows0, rows1, acc_sh, sem0, sem1, *,
                     feat):
    c = lax.axis_index("c")
    s = lax.axis_index("s")
    ytab = y_hbm
    pltpu.sync_copy(z_hbm, rows0)
    for r in range(RPT // CHUNK):
        pltpu.sync_copy(rows0, acc_sh.at[pl.ds(s * RPT + r * CHUNK, CHUNK)])
    plsc.subcore_barrier()

    def group(g, carry):
        pltpu.sync_copy(src_hbm.at[c, s, pl.ds(g * GRP, GRP)], idxs_v)
        pltpu.sync_copy(dst_hbm.at[c, s, pl.ds(g * GRP, GRP)], idxd_v)
        pltpu.async_copy(ytab.at[idxs_v.at[0]], rows0, sem0)
        pltpu.async_copy(ytab.at[idxs_v.at[1]], rows1, sem1)

        def pair(i, carry2):
            i0 = 2 * i
            nxt = jnp.minimum(i0 + 2, GRP - 1)
            nxt1 = jnp.minimum(i0 + 3, GRP - 1)
            pltpu.make_async_copy(ytab.at[idxs_v.at[0]], rows0, sem0).wait()
            pltpu.sync_copy(rows0, acc_sh.at[idxd_v.at[i0]], add=True)
            pltpu.async_copy(ytab.at[idxs_v.at[nxt]], rows0, sem0)
            pltpu.make_async_copy(ytab.at[idxs_v.at[0]], rows1, sem1).wait()
            pltpu.sync_copy(rows1, acc_sh.at[idxd_v.at[i0 + 1]], add=True)
            pltpu.async_copy(ytab.at[idxs_v.at[nxt1]], rows1, sem1)
            return carry2

        lax.fori_loop(0, GRP // 2, pair, 0)
        pltpu.make_async_copy(ytab.at[idxs_v.at[0]], rows0, sem0).wait()
        pltpu.make_async_copy(ytab.at[idxs_v.at[0]], rows1, sem1).wait()
        return carry

    lax.fori_loop(0, NCHUNK // GRP, group, 0)
    plsc.subcore_barrier()
    pltpu.sync_copy(acc_sh.at[pl.ds(s * RPT, RPT)],
                    out_hbm.at[pl.ds(c * NP + s * RPT, RPT)])


def _edge_aggregate_narrow(ys_flat, src_st, dst_st, feat):
    kern = pl.kernel(
        functools.partial(_agg_narrow_body, feat=feat),
        out_type=jax.ShapeDtypeStruct((2 * NP, feat), F32),
        mesh=_sc_mesh(),
        compiler_params=pltpu.CompilerParams(use_tc_tiling_on_sc=False),
        scratch_types=[
            pltpu.VMEM((GRP, CHUNK), jnp.int32),
            pltpu.VMEM((GRP, CHUNK), jnp.int32),
            pltpu.VMEM((CHUNK, feat), F32),
            pltpu.VMEM((CHUNK, feat), F32),
            pltpu.VMEM_SHARED((NP, feat), F32),
            pltpu.SemaphoreType.DMA,
            pltpu.SemaphoreType.DMA,
        ],
    )
    return kern(ys_flat, src_st, dst_st, jnp.zeros((CHUNK, feat), F32))


def _edge_aggregate(ys, src_st, dst_st, zero_rows, feat):
    kern = pl.kernel(
        _agg_body,
        out_type=jax.ShapeDtypeStruct((2 * NP, feat), F32),
        mesh=_sc_mesh(),
        scratch_types=[
            pltpu.VMEM((GRP, CHUNK), jnp.int32),
            pltpu.VMEM((GRP, CHUNK), jnp.int32),
            pltpu.VMEM((CHUNK, feat), F32),
            pltpu.VMEM((CHUNK, feat), F32),
            pltpu.VMEM_SHARED((NP, feat), F32),
            pltpu.SemaphoreType.DMA,
            pltpu.SemaphoreType.DMA,
        ],
    )
    return kern(ys, src_st, dst_st, zero_rows)


def _onehot_body(b_ref, m_ref):
    rows = lax.broadcasted_iota(jnp.int32, (B, NP), 0)
    m_ref[...] = jnp.where(rows == b_ref[...], 1.0, 0.0).astype(F32)


def _onehot(batch_pad):
    return pl.pallas_call(
        _onehot_body,
        out_shape=jax.ShapeDtypeStruct((B, NP), F32),
    )(batch_pad.reshape(1, NP))


def _y0_body(x_ref, w_ref, cnt_ref, y_ref):
    dis = lax.rsqrt(cnt_ref[:, :1] + 1.0)
    y_ref[...] = jnp.dot(x_ref[...], w_ref[...],
                         preferred_element_type=F32) * dis


def _y0(xs, W1, cnt):
    return pl.pallas_call(
        _y0_body,
        out_shape=jax.ShapeDtypeStruct((2 * NP, W1.shape[1]), F32),
    )(xs, W1, cnt)


def _level_body(agg_ref, y_ref, cnt_ref, b_ref, w1_ref, b1_ref, w2_ref, b2_ref,
                m_ref, wn_ref, pooled_ref, ynext_ref):
    dis = lax.rsqrt(cnt_ref[:, :1] + 1.0)
    h = jax.nn.relu((agg_ref[...] + y_ref[...]) * dis + b_ref[...])
    hid = jax.nn.relu(jnp.dot(h, w1_ref[...], preferred_element_type=F32)
                      + b1_ref[...])
    a = jnp.tanh(jnp.dot(hid, w2_ref[...], preferred_element_type=F32)
                 + b2_ref[...])
    xatt = a * h + h
    m = m_ref[...]
    cntg = jnp.sum(m, axis=1, keepdims=True)
    mean = jnp.dot(m, xatt, preferred_element_type=F32) / jnp.maximum(cntg, 1.0)
    tg = jnp.tanh(mean)
    tgb = lax.dot_general(m, tg, (((0,), (0,)), ((), ())),
                          preferred_element_type=F32)
    coefs = jax.nn.sigmoid(jnp.sum(xatt * tgb, axis=1, keepdims=True))
    pooled_ref[...] = jnp.dot(m, coefs * xatt, preferred_element_type=F32)
    if ynext_ref is not None:
        ynext_ref[...] = jnp.dot(h, wn_ref[...], preferred_element_type=F32) * dis


def _level(agg, y, cnt, b, w1, b1, w2, b2, m, wn):
    feat = y.shape[1]
    if wn is None:
        out_shape = [jax.ShapeDtypeStruct((B, feat), F32)]
        body = lambda *rs: _level_body(*rs[:9], None, rs[9], None)
        args = (agg, y, cnt, b, w1, b1, w2, b2, m)
    else:
        out_shape = [jax.ShapeDtypeStruct((B, feat), F32),
                     jax.ShapeDtypeStruct((NP, wn.shape[1]), F32)]
        body = _level_body
        args = (agg, y, cnt, b, w1, b1, w2, b2, m, wn)
    return pl.pallas_call(body, out_shape=out_shape)(*args)


def _head_body(p11, p12, p21, p22, p31, p32, w1, t1, c1, w2, t2, c2, w3, t3, c3,
               sw1, sb1, sw2, sb2, fw, fb, out_ref):
    def tnet(e1, e2, w_ref, wbt_ref, bias_ref, d):
        outer = (e1[:, :, None] * e2[:, None, :]).reshape(B, d * d)
        s = jnp.dot(outer, w_ref[...], preferred_element_type=F32)
        comb = jnp.concatenate([e1, e2], axis=1)
        return jax.nn.relu(s + jnp.dot(comb, wbt_ref[...],
                                       preferred_element_type=F32)
                           + bias_ref[...])

    s1 = tnet(p11[...], p12[...], w1, t1, c1, 128)
    s2 = tnet(p21[...], p22[...], w2, t2, c2, 64)
    s3 = tnet(p31[...], p32[...], w3, t3, c3, 32)
    scores = jnp.concatenate([s3, s2, s1], axis=1)
    se = jax.nn.sigmoid(
        jnp.dot(jax.nn.relu(jnp.dot(scores, sw1[...],
                                    preferred_element_type=F32) + sb1[...]),
                sw2[...], preferred_element_type=F32) + sb2[...])
    out_ref[...] = jax.nn.relu(
        jnp.dot(se * scores + scores, fw[...], preferred_element_type=F32)
        + fb[...])


def _head(pools, tn, se_w1, se_b1, se_w2, se_b2, fc_w, fc_b):
    args = list(pools)
    for w2d, wbt, bias in tn:
        args += [w2d, wbt, bias]
    args += [se_w1, se_b1.reshape(1, -1), se_w2, se_b2.reshape(1, -1),
             fc_w, fc_b.reshape(1, -1)]
    return pl.pallas_call(
        _head_body,
        out_shape=jax.ShapeDtypeStruct((B, 64), F32),
    )(*args)


def kernel(x1, x2, edge_index1, edge_index2, batch1, batch2, W1, b1, W2, b2,
           W3, b3, a1_w1, a1_b1, a1_w2, a1_b2, a2_w1, a2_b1, a2_w2, a2_b2,
           a3_w1, a3_b1, a3_w2, a3_b2, tn1_W, tn1_Wb, tn1_bias,
           tn2_W, tn2_Wb, tn2_bias, tn3_W, tn3_Wb, tn3_bias,
           se_w1, se_b1, se_w2, se_b2, fc_w, fc_b):
    pad_e = EP - E
    pad_n = NP - N

    def pad_edges(v, off):
        return jnp.concatenate([v + off, jnp.full((pad_e,), off + NP - 1,
                                                  jnp.int32)])

    src_st = jnp.concatenate([pad_edges(edge_index1[0], 0),
                              pad_edges(edge_index2[0], NP)]
                             ).reshape(2, NTILE, NCHUNK, CHUNK)
    dst_st = jnp.concatenate([pad_edges(edge_index1[1], 0),
                              pad_edges(edge_index2[1], 0)]
                             ).reshape(2, NTILE, NCHUNK, CHUNK)
    xs = jnp.concatenate([
        jnp.pad(x1, ((0, pad_n), (0, 0))),
        jnp.pad(x2, ((0, pad_n), (0, 0)))])
    bp1 = jnp.pad(batch1, (0, pad_n), constant_values=jnp.int32(B + 7))
    bp2 = jnp.pad(batch2, (0, pad_n), constant_values=jnp.int32(B + 7))

    unit_rows = jnp.concatenate([jnp.zeros((CHUNK, 16), F32),
                                 jnp.ones((CHUNK, 16), F32)])

    cnt = _degree_count(dst_st, unit_rows)
    m1 = _onehot(bp1)
    m2 = _onehot(bp2)

    ys = _y0(xs, W1, cnt)
    pools = []
    att = [(a1_w1, a1_b1, a1_w2, a1_b2), (a2_w1, a2_b1, a2_w2, a2_b2),
           (a3_w1, a3_b1, a3_w2, a3_b2)]
    biases = [b1, b2, b3]
    nexts = [W2, W3, None]
    zero_rows = jnp.zeros((CHUNK, 128), F32)
    for lvl in range(3):
        feat = biases[lvl].shape[0]
        if lvl == 0:
            agg = _edge_aggregate(ys, src_st, dst_st, zero_rows, 128)
        else:
            agg = _edge_aggregate_narrow(ys, src_st, dst_st, feat)
        w1a, b1a, w2a, b2a = att[lvl]
        bl = biases[lvl].reshape(1, feat)
        outs = []
        for g, m in ((0, m1), (1, m2)):
            res = _level(agg[g * NP:(g + 1) * NP, :feat],
                         ys[g * NP:(g + 1) * NP, :feat],
                         cnt[g * NP:(g + 1) * NP], bl,
                         w1a, b1a.reshape(1, -1), w2a, b2a.reshape(1, -1),
                         m, nexts[lvl])
            outs.append(res)
        pools.append((outs[0][0], outs[1][0]))
        if nexts[lvl] is not None:
            ys = jnp.concatenate([outs[0][1], outs[1][1]])

    tn = [(tn1_W.reshape(128 * 128, 64), tn1_Wb.T, tn1_bias.reshape(1, -1)),
          (tn2_W.reshape(64 * 64, 32), tn2_Wb.T, tn2_bias.reshape(1, -1)),
          (tn3_W.reshape(32 * 32, 16), tn3_Wb.T, tn3_bias.reshape(1, -1))]
    pool_args = [pools[0][0], pools[0][1], pools[1][0], pools[1][1],
                 pools[2][0], pools[2][1]]
    return _head(pool_args, tn, se_w1, se_b1, se_w2, se_b2, fc_w, fc_b)


# speedup vs baseline: 1.3387x; 1.0066x over previous
import functools

import jax
import jax.numpy as jnp
from jax import lax
from jax.experimental import pallas as pl
from jax.experimental.pallas import tpu as pltpu
from jax.experimental.pallas import tpu_sc as plsc

N = 10000
NP = 10240
E = 320000
NTILE = 16
CHUNK = 128
NCHUNK = 160
GRP = 32
EPT = NCHUNK * CHUNK
EP = EPT * NTILE
RPT = NP // NTILE
B = 128
F32 = jnp.float32


def _sc_mesh():
    return plsc.VectorSubcoreMesh(core_axis_name="c", subcore_axis_name="s")


def _deg_body(dst_hbm, u_hbm, out_hbm, unit_v, idx_v, hist_sh, sem):
    c = lax.axis_index("c")
    s = lax.axis_index("s")
    pltpu.sync_copy(u_hbm.at[pl.ds(0, CHUNK)], unit_v)
    for r in range(RPT // CHUNK):
        pltpu.sync_copy(unit_v, hist_sh.at[pl.ds(s * RPT + r * CHUNK, CHUNK)])
    plsc.subcore_barrier()
    pltpu.sync_copy(u_hbm.at[pl.ds(CHUNK, CHUNK)], unit_v)
    pltpu.sync_copy(dst_hbm.at[c, s], idx_v)

    def chunk(i, carry):
        pltpu.sync_copy(unit_v, hist_sh.at[idx_v.at[i]], add=True)
        return carry

    lax.fori_loop(0, NCHUNK, chunk, 0)
    plsc.subcore_barrier()
    pltpu.sync_copy(hist_sh.at[pl.ds(s * RPT, RPT)],
                    out_hbm.at[pl.ds(c * NP + s * RPT, RPT)])


def _degree_count(dst_st, unit_rows):
    kern = pl.kernel(
        _deg_body,
        out_type=jax.ShapeDtypeStruct((2 * NP, 16), F32),
        mesh=_sc_mesh(),
        scratch_types=[
            pltpu.VMEM((CHUNK, 16), F32),
            pltpu.VMEM((NCHUNK, CHUNK), jnp.int32),
            pltpu.VMEM_SHARED((NP, 16), F32),
            pltpu.SemaphoreType.DMA,
        ],
    )
    return kern(dst_st, unit_rows)


def _agg_body(y_hbm, src_hbm, dst_hbm, z_hbm, out_hbm,
              idxs_v, idxd_v, rows0, rows1, acc_sh, sem0, sem1):
    c = lax.axis_index("c")
    s = lax.axis_index("s")
    pltpu.sync_copy(z_hbm, rows0)
    for r in range(RPT // CHUNK):
        pltpu.sync_copy(rows0, acc_sh.at[pl.ds(s * RPT + r * CHUNK, CHUNK)])
    plsc.subcore_barrier()
    def group(g, carry):
        pltpu.sync_copy(src_hbm.at[c, s, pl.ds(g * GRP, GRP)], idxs_v)
        pltpu.sync_copy(dst_hbm.at[c, s, pl.ds(g * GRP, GRP)], idxd_v)
        pltpu.async_copy(y_hbm.at[idxs_v.at[0]], rows0, sem0)
        pltpu.async_copy(y_hbm.at[idxs_v.at[1]], rows1, sem1)

        def pair(i, carry2):
            i0 = 2 * i
            nxt = jnp.minimum(i0 + 2, GRP - 1)
            nxt1 = jnp.minimum(i0 + 3, GRP - 1)
            pltpu.make_async_copy(y_hbm.at[idxs_v.at[0]], rows0, sem0).wait()
            pltpu.sync_copy(rows0, acc_sh.at[idxd_v.at[i0]], add=True)
            pltpu.async_copy(y_hbm.at[idxs_v.at[nxt]], rows0, sem0)
            pltpu.make_async_copy(y_hbm.at[idxs_v.at[0]], rows1, sem1).wait()
            pltpu.sync_copy(rows1, acc_sh.at[idxd_v.at[i0 + 1]], add=True)
            pltpu.async_copy(y_hbm.at[idxs_v.at[nxt1]], rows1, sem1)
            return carry2

        lax.fori_loop(0, GRP // 2, pair, 0)
        pltpu.make_async_copy(y_hbm.at[idxs_v.at[0]], rows0, sem0).wait()
        pltpu.make_async_copy(y_hbm.at[idxs_v.at[0]], rows1, sem1).wait()
        return carry

    lax.fori_loop(0, NCHUNK // GRP, group, 0)
    plsc.subcore_barrier()
    pltpu.sync_copy(acc_sh.at[pl.ds(s * RPT, RPT)],
                    out_hbm.at[pl.ds(c * NP + s * RPT, RPT)])


def _agg_narrow_body(y_hbm, src_hbm, dst_hbm, z_hbm, out_hbm,
                     idxs_v, idxd_v, rows0, rows1, acc_sh, sem0, sem1, *,
                     feat):
    c = lax.axis_index("c")
    s = lax.axis_index("s")
    ytab = y_hbm
    pltpu.sync_copy(z_hbm, rows0)
    for r in range(RPT // CHUNK):
        pltpu.sync_copy(rows0, acc_sh.at[pl.ds(s * RPT + r * CHUNK, CHUNK)])
    plsc.subcore_barrier()

    def group(g, carry):
        pltpu.sync_copy(src_hbm.at[c, s, pl.ds(g * GRP, GRP)], idxs_v)
        pltpu.sync_copy(dst_hbm.at[c, s, pl.ds(g * GRP, GRP)], idxd_v)
        pltpu.async_copy(ytab.at[idxs_v.at[0]], rows0, sem0)
        pltpu.async_copy(ytab.at[idxs_v.at[1]], rows1, sem1)

        def pair(i, carry2):
            i0 = 2 * i
            nxt = jnp.minimum(i0 + 2, GRP - 1)
            nxt1 = jnp.minimum(i0 + 3, GRP - 1)
            pltpu.make_async_copy(ytab.at[idxs_v.at[0]], rows0, sem0).wait()
            pltpu.sync_copy(rows0, acc_sh.at[idxd_v.at[i0]], add=True)
            pltpu.async_copy(ytab.at[idxs_v.at[nxt]], rows0, sem0)
            pltpu.make_async_copy(ytab.at[idxs_v.at[0]], rows1, sem1).wait()
            pltpu.sync_copy(rows1, acc_sh.at[idxd_v.at[i0 + 1]], add=True)
            pltpu.async_copy(ytab.at[idxs_v.at[nxt1]], rows1, sem1)
            return carry2

        lax.fori_loop(0, GRP // 2, pair, 0)
        pltpu.make_async_copy(ytab.at[idxs_v.at[0]], rows0, sem0).wait()
        pltpu.make_async_copy(ytab.at[idxs_v.at[0]], rows1, sem1).wait()
        return carry

    lax.fori_loop(0, NCHUNK // GRP, group, 0)
    plsc.subcore_barrier()
    pltpu.sync_copy(acc_sh.at[pl.ds(s * RPT, RPT)],
                    out_hbm.at[pl.ds(c * NP + s * RPT, RPT)])


def _edge_aggregate_narrow(ys_flat, src_st, dst_st, feat):
    kern = pl.kernel(
        functools.partial(_agg_narrow_body, feat=feat),
        out_type=jax.ShapeDtypeStruct((2 * NP, feat), F32),
        mesh=_sc_mesh(),
        compiler_params=pltpu.CompilerParams(use_tc_tiling_on_sc=False),
        scratch_types=[
            pltpu.VMEM((GRP, CHUNK), jnp.int32),
            pltpu.VMEM((GRP, CHUNK), jnp.int32),
            pltpu.VMEM((CHUNK, feat), F32),
            pltpu.VMEM((CHUNK, feat), F32),
            pltpu.VMEM_SHARED((NP, feat), F32),
            pltpu.SemaphoreType.DMA,
            pltpu.SemaphoreType.DMA,
        ],
    )
    return kern(ys_flat, src_st, dst_st, jnp.zeros((CHUNK, feat), F32))


def _edge_aggregate(ys, src_st, dst_st, zero_rows, feat):
    kern = pl.kernel(
        _agg_body,
        out_type=jax.ShapeDtypeStruct((2 * NP, feat), F32),
        mesh=_sc_mesh(),
        scratch_types=[
            pltpu.VMEM((GRP, CHUNK), jnp.int32),
            pltpu.VMEM((GRP, CHUNK), jnp.int32),
            pltpu.VMEM((CHUNK, feat), F32),
            pltpu.VMEM((CHUNK, feat), F32),
            pltpu.VMEM_SHARED((NP, feat), F32),
            pltpu.SemaphoreType.DMA,
            pltpu.SemaphoreType.DMA,
        ],
    )
    return kern(ys, src_st, dst_st, zero_rows)


def _onehot_body(b_ref, m_ref):
    rows = lax.broadcasted_iota(jnp.int32, (B, NP), 0)
    m_ref[...] = jnp.where(rows == b_ref[...], 1.0, 0.0).astype(F32)


def _onehot(batch_pad):
    return pl.pallas_call(
        _onehot_body,
        out_shape=jax.ShapeDtypeStruct((B, NP), F32),
    )(batch_pad.reshape(1, NP))


def _y0_body(x_ref, w_ref, cnt_ref, y_ref):
    dis = lax.rsqrt(cnt_ref[:, :1] + 1.0)
    y_ref[...] = jnp.dot(x_ref[...], w_ref[...],
                         preferred_element_type=F32) * dis


def _y0(xs, W1, cnt):
    return pl.pallas_call(
        _y0_body,
        out_shape=jax.ShapeDtypeStruct((2 * NP, W1.shape[1]), F32),
    )(xs, W1, cnt)


def _hy_body(agg_ref, y_ref, cnt_ref, b_ref, wn_ref, h_ref, ynext_ref):
    dis = lax.rsqrt(cnt_ref[:, :1] + 1.0)
    h = jax.nn.relu((agg_ref[...] + y_ref[...]) * dis + b_ref[...])
    h_ref[...] = h
    if ynext_ref is not None:
        ynext_ref[...] = jnp.dot(h, wn_ref[...], preferred_element_type=F32) * dis


def _hy(agg, y, cnt, b, wn):
    feat = y.shape[1]
    if wn is None:
        out_shape = [jax.ShapeDtypeStruct((NP, feat), F32)]
        body = lambda *rs: _hy_body(*rs[:4], None, rs[4], None)
        args = (agg, y, cnt, b)
    else:
        out_shape = [jax.ShapeDtypeStruct((NP, feat), F32),
                     jax.ShapeDtypeStruct((NP, wn.shape[1]), F32)]
        body = _hy_body
        args = (agg, y, cnt, b, wn)
    return pl.pallas_call(body, out_shape=out_shape)(*args)


def _pool_body(h_ref, w1_ref, b1_ref, w2_ref, b2_ref, m_ref, pooled_ref):
    h = h_ref[...]
    hid = jax.nn.relu(jnp.dot(h, w1_ref[...], preferred_element_type=F32)
                      + b1_ref[...])
    a = jnp.tanh(jnp.dot(hid, w2_ref[...], preferred_element_type=F32)
                 + b2_ref[...])
    xatt = a * h + h
    m = m_ref[...]
    cntg = jnp.sum(m, axis=1, keepdims=True)
    mean = jnp.dot(m, xatt, preferred_element_type=F32) / jnp.maximum(cntg, 1.0)
    tg = jnp.tanh(mean)
    tgb = lax.dot_general(m, tg, (((0,), (0,)), ((), ())),
                          preferred_element_type=F32)
    coefs = jax.nn.sigmoid(jnp.sum(xatt * tgb, axis=1, keepdims=True))
    pooled_ref[...] = jnp.dot(m, coefs * xatt, preferred_element_type=F32)


def _pool(h, w1, b1, w2, b2, m):
    feat = h.shape[1]
    return pl.pallas_call(
        _pool_body,
        out_shape=jax.ShapeDtypeStruct((B, feat), F32),
    )(h, w1, b1, w2, b2, m)


def _head_body(p11, p12, p21, p22, p31, p32, w1, t1, c1, w2, t2, c2, w3, t3, c3,
               sw1, sb1, sw2, sb2, fw, fb, out_ref):
    def tnet(e1, e2, w_ref, wbt_ref, bias_ref, d):
        outer = (e1[:, :, None] * e2[:, None, :]).reshape(B, d * d)
        s = jnp.dot(outer, w_ref[...], preferred_element_type=F32)
        comb = jnp.concatenate([e1, e2], axis=1)
        return jax.nn.relu(s + jnp.dot(comb, wbt_ref[...],
                                       preferred_element_type=F32)
                           + bias_ref[...])

    s1 = tnet(p11[...], p12[...], w1, t1, c1, 128)
    s2 = tnet(p21[...], p22[...], w2, t2, c2, 64)
    s3 = tnet(p31[...], p32[...], w3, t3, c3, 32)
    scores = jnp.concatenate([s3, s2, s1], axis=1)
    se = jax.nn.sigmoid(
        jnp.dot(jax.nn.relu(jnp.dot(scores, sw1[...],
                                    preferred_element_type=F32) + sb1[...]),
                sw2[...], preferred_element_type=F32) + sb2[...])
    out_ref[...] = jax.nn.relu(
        jnp.dot(se * scores + scores, fw[...], preferred_element_type=F32)
        + fb[...])


def _head(pools, tn, se_w1, se_b1, se_w2, se_b2, fc_w, fc_b):
    args = list(pools)
    for w2d, wbt, bias in tn:
        args += [w2d, wbt, bias]
    args += [se_w1, se_b1.reshape(1, -1), se_w2, se_b2.reshape(1, -1),
             fc_w, fc_b.reshape(1, -1)]
    return pl.pallas_call(
        _head_body,
        out_shape=jax.ShapeDtypeStruct((B, 64), F32),
    )(*args)


def kernel(x1, x2, edge_index1, edge_index2, batch1, batch2, W1, b1, W2, b2,
           W3, b3, a1_w1, a1_b1, a1_w2, a1_b2, a2_w1, a2_b1, a2_w2, a2_b2,
           a3_w1, a3_b1, a3_w2, a3_b2, tn1_W, tn1_Wb, tn1_bias,
           tn2_W, tn2_Wb, tn2_bias, tn3_W, tn3_Wb, tn3_bias,
           se_w1, se_b1, se_w2, se_b2, fc_w, fc_b):
    pad_e = EP - E
    pad_n = NP - N

    def pad_edges(v, off):
        return jnp.concatenate([v + off, jnp.full((pad_e,), off + NP - 1,
                                                  jnp.int32)])

    src_st = jnp.concatenate([pad_edges(edge_index1[0], 0),
                              pad_edges(edge_index2[0], NP)]
                             ).reshape(2, NTILE, NCHUNK, CHUNK)
    dst_st = jnp.concatenate([pad_edges(edge_index1[1], 0),
                              pad_edges(edge_index2[1], 0)]
                             ).reshape(2, NTILE, NCHUNK, CHUNK)
    xs = jnp.concatenate([
        jnp.pad(x1, ((0, pad_n), (0, 0))),
        jnp.pad(x2, ((0, pad_n), (0, 0)))])
    bp1 = jnp.pad(batch1, (0, pad_n), constant_values=jnp.int32(B + 7))
    bp2 = jnp.pad(batch2, (0, pad_n), constant_values=jnp.int32(B + 7))

    unit_rows = jnp.concatenate([jnp.zeros((CHUNK, 16), F32),
                                 jnp.ones((CHUNK, 16), F32)])

    cnt = _degree_count(dst_st, unit_rows)
    m1 = _onehot(bp1)
    m2 = _onehot(bp2)

    ys = _y0(xs, W1, cnt)
    pools = []
    att = [(a1_w1, a1_b1, a1_w2, a1_b2), (a2_w1, a2_b1, a2_w2, a2_b2),
           (a3_w1, a3_b1, a3_w2, a3_b2)]
    biases = [b1, b2, b3]
    nexts = [W2, W3, None]
    zero_rows = jnp.zeros((CHUNK, 128), F32)
    for lvl in range(3):
        feat = biases[lvl].shape[0]
        if lvl == 0:
            agg = _edge_aggregate(ys, src_st, dst_st, zero_rows, 128)
        else:
            agg = _edge_aggregate_narrow(ys, src_st, dst_st, feat)
        w1a, b1a, w2a, b2a = att[lvl]
        bl = biases[lvl].reshape(1, feat)
        hs, yn = [], []
        for g in (0, 1):
            res = _hy(agg[g * NP:(g + 1) * NP, :feat],
                      ys[g * NP:(g + 1) * NP, :feat],
                      cnt[g * NP:(g + 1) * NP], bl, nexts[lvl])
            hs.append(res[0])
            if nexts[lvl] is not None:
                yn.append(res[1])
        if nexts[lvl] is not None:
            ys = jnp.concatenate(yn)
        pools.append((_pool(hs[0], w1a, b1a.reshape(1, -1), w2a,
                            b2a.reshape(1, -1), m1),
                      _pool(hs[1], w1a, b1a.reshape(1, -1), w2a,
                            b2a.reshape(1, -1), m2)))

    tn = [(tn1_W.reshape(128 * 128, 64), tn1_Wb.T, tn1_bias.reshape(1, -1)),
          (tn2_W.reshape(64 * 64, 32), tn2_Wb.T, tn2_bias.reshape(1, -1)),
          (tn3_W.reshape(32 * 32, 16), tn3_Wb.T, tn3_bias.reshape(1, -1))]
    pool_args = [pools[0][0], pools[0][1], pools[1][0], pools[1][1],
                 pools[2][0], pools[2][1]]
    return _head(pool_args, tn, se_w1, se_b1, se_w2, se_b2, fc_w, fc_b)


# speedup vs baseline: 1.3835x; 1.0334x over previous
import functools

import jax
import jax.numpy as jnp
from jax import lax
from jax.experimental import pallas as pl
from jax.experimental.pallas import tpu as pltpu
from jax.experimental.pallas import tpu_sc as plsc

N = 10000
NP = 10240
E = 320000
NTILE = 16
CHUNK = 128
NCHUNK = 160
GRP = 32
EPT = NCHUNK * CHUNK
EP = EPT * NTILE
RPT = NP // NTILE
B = 128
F32 = jnp.float32


def _sc_mesh():
    return plsc.VectorSubcoreMesh(core_axis_name="c", subcore_axis_name="s")


def _deg_body(dst_hbm, u_hbm, out_hbm, unit_v, idx_v, hist_sh, sem):
    c = lax.axis_index("c")
    s = lax.axis_index("s")
    pltpu.sync_copy(u_hbm.at[pl.ds(0, CHUNK)], unit_v)
    for r in range(RPT // CHUNK):
        pltpu.sync_copy(unit_v, hist_sh.at[pl.ds(s * RPT + r * CHUNK, CHUNK)])
    plsc.subcore_barrier()
    pltpu.sync_copy(u_hbm.at[pl.ds(CHUNK, CHUNK)], unit_v)
    pltpu.sync_copy(dst_hbm.at[c, s], idx_v)

    def chunk(i, carry):
        pltpu.sync_copy(unit_v, hist_sh.at[idx_v.at[i]], add=True)
        return carry

    lax.fori_loop(0, NCHUNK, chunk, 0)
    plsc.subcore_barrier()
    pltpu.sync_copy(hist_sh.at[pl.ds(s * RPT, RPT)],
                    out_hbm.at[pl.ds(c * NP + s * RPT, RPT)])


def _degree_count(dst_st, unit_rows):
    kern = pl.kernel(
        _deg_body,
        out_type=jax.ShapeDtypeStruct((2 * NP, 16), F32),
        mesh=_sc_mesh(),
        scratch_types=[
            pltpu.VMEM((CHUNK, 16), F32),
            pltpu.VMEM((NCHUNK, CHUNK), jnp.int32),
            pltpu.VMEM_SHARED((NP, 16), F32),
            pltpu.SemaphoreType.DMA,
        ],
    )
    return kern(dst_st, unit_rows)


def _agg_body(y_hbm, src_hbm, dst_hbm, z_hbm, out_hbm,
              idxs_v, idxd_v, rows0, rows1, acc_sh, sem0, sem1):
    c = lax.axis_index("c")
    s = lax.axis_index("s")
    pltpu.sync_copy(z_hbm, rows0)
    for r in range(RPT // CHUNK):
        pltpu.sync_copy(rows0, acc_sh.at[pl.ds(s * RPT + r * CHUNK, CHUNK)])
    plsc.subcore_barrier()
    def group(g, carry):
        pltpu.sync_copy(src_hbm.at[c, s, pl.ds(g * GRP, GRP)], idxs_v)
        pltpu.sync_copy(dst_hbm.at[c, s, pl.ds(g * GRP, GRP)], idxd_v)
        pltpu.async_copy(y_hbm.at[idxs_v.at[0]], rows0, sem0)
        pltpu.async_copy(y_hbm.at[idxs_v.at[1]], rows1, sem1)

        def pair(i, carry2):
            i0 = 2 * i
            nxt = jnp.minimum(i0 + 2, GRP - 1)
            nxt1 = jnp.minimum(i0 + 3, GRP - 1)
            pltpu.make_async_copy(y_hbm.at[idxs_v.at[0]], rows0, sem0).wait()
            pltpu.sync_copy(rows0, acc_sh.at[idxd_v.at[i0]], add=True)
            pltpu.async_copy(y_hbm.at[idxs_v.at[nxt]], rows0, sem0)
            pltpu.make_async_copy(y_hbm.at[idxs_v.at[0]], rows1, sem1).wait()
            pltpu.sync_copy(rows1, acc_sh.at[idxd_v.at[i0 + 1]], add=True)
            pltpu.async_copy(y_hbm.at[idxs_v.at[nxt1]], rows1, sem1)
            return carry2

        lax.fori_loop(0, GRP // 2, pair, 0)
        pltpu.make_async_copy(y_hbm.at[idxs_v.at[0]], rows0, sem0).wait()
        pltpu.make_async_copy(y_hbm.at[idxs_v.at[0]], rows1, sem1).wait()
        return carry

    lax.fori_loop(0, NCHUNK // GRP, group, 0)
    plsc.subcore_barrier()
    pltpu.sync_copy(acc_sh.at[pl.ds(s * RPT, RPT)],
                    out_hbm.at[pl.ds(c * NP + s * RPT, RPT)])


def _agg_narrow_body(y_hbm, src_hbm, dst_hbm, z_hbm, out_hbm,
                     idxs_v, idxd_v, rows0, rows1, acc_sh, sem0, sem1, *,
                     feat):
    c = lax.axis_index("c")
    s = lax.axis_index("s")
    ytab = y_hbm
    pltpu.sync_copy(z_hbm, rows0)
    for r in range(RPT // CHUNK):
        pltpu.sync_copy(rows0, acc_sh.at[pl.ds(s * RPT + r * CHUNK, CHUNK)])
    plsc.subcore_barrier()

    def group(g, carry):
        pltpu.sync_copy(src_hbm.at[c, s, pl.ds(g * GRP, GRP)], idxs_v)
        pltpu.sync_copy(dst_hbm.at[c, s, pl.ds(g * GRP, GRP)], idxd_v)
        pltpu.async_copy(ytab.at[idxs_v.at[0]], rows0, sem0)
        pltpu.async_copy(ytab.at[idxs_v.at[1]], rows1, sem1)

        def pair(i, carry2):
            i0 = 2 * i
            nxt = jnp.minimum(i0 + 2, GRP - 1)
            nxt1 = jnp.minimum(i0 + 3, GRP - 1)
            pltpu.make_async_copy(ytab.at[idxs_v.at[0]], rows0, sem0).wait()
            pltpu.sync_copy(rows0, acc_sh.at[idxd_v.at[i0]], add=True)
            pltpu.async_copy(ytab.at[idxs_v.at[nxt]], rows0, sem0)
            pltpu.make_async_copy(ytab.at[idxs_v.at[0]], rows1, sem1).wait()
            pltpu.sync_copy(rows1, acc_sh.at[idxd_v.at[i0 + 1]], add=True)
            pltpu.async_copy(ytab.at[idxs_v.at[nxt1]], rows1, sem1)
            return carry2

        lax.fori_loop(0, GRP // 2, pair, 0)
        pltpu.make_async_copy(ytab.at[idxs_v.at[0]], rows0, sem0).wait()
        pltpu.make_async_copy(ytab.at[idxs_v.at[0]], rows1, sem1).wait()
        return carry

    lax.fori_loop(0, NCHUNK // GRP, group, 0)
    plsc.subcore_barrier()
    pltpu.sync_copy(acc_sh.at[pl.ds(s * RPT, RPT)],
                    out_hbm.at[pl.ds(c * NP + s * RPT, RPT)])


def _edge_aggregate_narrow(ys_flat, src_st, dst_st, feat):
    kern = pl.kernel(
        functools.partial(_agg_narrow_body, feat=feat),
        out_type=jax.ShapeDtypeStruct((2 * NP, feat), F32),
        mesh=_sc_mesh(),
        compiler_params=pltpu.CompilerParams(use_tc_tiling_on_sc=False),
        scratch_types=[
            pltpu.VMEM((GRP, CHUNK), jnp.int32),
            pltpu.VMEM((GRP, CHUNK), jnp.int32),
            pltpu.VMEM((CHUNK, feat), F32),
            pltpu.VMEM((CHUNK, feat), F32),
            pltpu.VMEM_SHARED((NP, feat), F32),
            pltpu.SemaphoreType.DMA,
            pltpu.SemaphoreType.DMA,
        ],
    )
    return kern(ys_flat, src_st, dst_st, jnp.zeros((CHUNK, feat), F32))


def _edge_aggregate(ys, src_st, dst_st, zero_rows, feat):
    kern = pl.kernel(
        _agg_body,
        out_type=jax.ShapeDtypeStruct((2 * NP, feat), F32),
        mesh=_sc_mesh(),
        scratch_types=[
            pltpu.VMEM((GRP, CHUNK), jnp.int32),
            pltpu.VMEM((GRP, CHUNK), jnp.int32),
            pltpu.VMEM((CHUNK, feat), F32),
            pltpu.VMEM((CHUNK, feat), F32),
            pltpu.VMEM_SHARED((NP, feat), F32),
            pltpu.SemaphoreType.DMA,
            pltpu.SemaphoreType.DMA,
        ],
    )
    return kern(ys, src_st, dst_st, zero_rows)


def _onehot_body(b_ref, m1_ref, m2_ref):
    rows = lax.broadcasted_iota(jnp.int32, (B, NP), 0)
    m1_ref[...] = jnp.where(rows == b_ref[0:1, :], 1.0, 0.0).astype(F32)
    m2_ref[...] = jnp.where(rows == b_ref[1:2, :], 1.0, 0.0).astype(F32)


def _onehot(batch_pad2):
    return pl.pallas_call(
        _onehot_body,
        out_shape=[jax.ShapeDtypeStruct((B, NP), F32),
                   jax.ShapeDtypeStruct((B, NP), F32)],
    )(batch_pad2)


def _y0_body(x_ref, w_ref, cnt_ref, y_ref):
    dis = lax.rsqrt(cnt_ref[:, :1] + 1.0)
    y_ref[...] = jnp.dot(x_ref[...], w_ref[...],
                         preferred_element_type=F32) * dis


def _y0(xs, W1, cnt):
    return pl.pallas_call(
        _y0_body,
        out_shape=jax.ShapeDtypeStruct((2 * NP, W1.shape[1]), F32),
    )(xs, W1, cnt)


def _hy_body(agg_ref, y_ref, cnt_ref, b_ref, wn_ref, h_ref, ynext_ref):
    dis = lax.rsqrt(cnt_ref[:, :1] + 1.0)
    h = jax.nn.relu((agg_ref[...] + y_ref[...]) * dis + b_ref[...])
    h_ref[...] = h
    if ynext_ref is not None:
        ynext_ref[...] = jnp.dot(h, wn_ref[...], preferred_element_type=F32) * dis


def _hy(agg, y, cnt, b, wn):
    feat = y.shape[1]
    if wn is None:
        out_shape = [jax.ShapeDtypeStruct((2 * NP, feat), F32)]
        body = lambda *rs: _hy_body(*rs[:4], None, rs[4], None)
        args = (agg, y, cnt, b)
    else:
        out_shape = [jax.ShapeDtypeStruct((2 * NP, feat), F32),
                     jax.ShapeDtypeStruct((2 * NP, wn.shape[1]), F32)]
        body = _hy_body
        args = (agg, y, cnt, b, wn)
    return pl.pallas_call(body, out_shape=out_shape)(*args)


def _pool_body(h_ref, w1_ref, b1_ref, w2_ref, b2_ref, m1_ref, m2_ref,
               p1_ref, p2_ref):
    h = h_ref[...]
    hid = jax.nn.relu(jnp.dot(h, w1_ref[...], preferred_element_type=F32)
                      + b1_ref[...])
    a = jnp.tanh(jnp.dot(hid, w2_ref[...], preferred_element_type=F32)
                 + b2_ref[...])
    xatt = a * h + h
    for g, (m_ref, p_ref) in enumerate(((m1_ref, p1_ref), (m2_ref, p2_ref))):
        xg = xatt[g * NP:(g + 1) * NP]
        m = m_ref[...]
        cntg = jnp.sum(m, axis=1, keepdims=True)
        mean = (jnp.dot(m, xg, preferred_element_type=F32)
                / jnp.maximum(cntg, 1.0))
        tg = jnp.tanh(mean)
        tgb = lax.dot_general(m, tg, (((0,), (0,)), ((), ())),
                              preferred_element_type=F32)
        coefs = jax.nn.sigmoid(jnp.sum(xg * tgb, axis=1, keepdims=True))
        p_ref[...] = jnp.dot(m, coefs * xg, preferred_element_type=F32)


def _pool(h, w1, b1, w2, b2, m1, m2):
    feat = h.shape[1]
    return pl.pallas_call(
        _pool_body,
        out_shape=[jax.ShapeDtypeStruct((B, feat), F32),
                   jax.ShapeDtypeStruct((B, feat), F32)],
    )(h, w1, b1, w2, b2, m1, m2)


def _head_body(p11, p12, p21, p22, p31, p32, w1, t1, c1, w2, t2, c2, w3, t3, c3,
               sw1, sb1, sw2, sb2, fw, fb, out_ref):
    def tnet(e1, e2, w_ref, wbt_ref, bias_ref, d):
        outer = (e1[:, :, None] * e2[:, None, :]).reshape(B, d * d)
        s = jnp.dot(outer, w_ref[...], preferred_element_type=F32)
        comb = jnp.concatenate([e1, e2], axis=1)
        return jax.nn.relu(s + jnp.dot(comb, wbt_ref[...],
                                       preferred_element_type=F32)
                           + bias_ref[...])

    s1 = tnet(p11[...], p12[...], w1, t1, c1, 128)
    s2 = tnet(p21[...], p22[...], w2, t2, c2, 64)
    s3 = tnet(p31[...], p32[...], w3, t3, c3, 32)
    scores = jnp.concatenate([s3, s2, s1], axis=1)
    se = jax.nn.sigmoid(
        jnp.dot(jax.nn.relu(jnp.dot(scores, sw1[...],
                                    preferred_element_type=F32) + sb1[...]),
                sw2[...], preferred_element_type=F32) + sb2[...])
    out_ref[...] = jax.nn.relu(
        jnp.dot(se * scores + scores, fw[...], preferred_element_type=F32)
        + fb[...])


def _head(pools, tn, se_w1, se_b1, se_w2, se_b2, fc_w, fc_b):
    args = list(pools)
    for w2d, wbt, bias in tn:
        args += [w2d, wbt, bias]
    args += [se_w1, se_b1.reshape(1, -1), se_w2, se_b2.reshape(1, -1),
             fc_w, fc_b.reshape(1, -1)]
    return pl.pallas_call(
        _head_body,
        out_shape=jax.ShapeDtypeStruct((B, 64), F32),
    )(*args)


def kernel(x1, x2, edge_index1, edge_index2, batch1, batch2, W1, b1, W2, b2,
           W3, b3, a1_w1, a1_b1, a1_w2, a1_b2, a2_w1, a2_b1, a2_w2, a2_b2,
           a3_w1, a3_b1, a3_w2, a3_b2, tn1_W, tn1_Wb, tn1_bias,
           tn2_W, tn2_Wb, tn2_bias, tn3_W, tn3_Wb, tn3_bias,
           se_w1, se_b1, se_w2, se_b2, fc_w, fc_b):
    pad_e = EP - E
    pad_n = NP - N

    def pad_edges(v, off):
        return jnp.concatenate([v + off, jnp.full((pad_e,), off + NP - 1,
                                                  jnp.int32)])

    src_st = jnp.concatenate([pad_edges(edge_index1[0], 0),
                              pad_edges(edge_index2[0], NP)]
                             ).reshape(2, NTILE, NCHUNK, CHUNK)
    dst_st = jnp.concatenate([pad_edges(edge_index1[1], 0),
                              pad_edges(edge_index2[1], 0)]
                             ).reshape(2, NTILE, NCHUNK, CHUNK)
    xs = jnp.concatenate([
        jnp.pad(x1, ((0, pad_n), (0, 0))),
        jnp.pad(x2, ((0, pad_n), (0, 0)))])
    bp1 = jnp.pad(batch1, (0, pad_n), constant_values=jnp.int32(B + 7))
    bp2 = jnp.pad(batch2, (0, pad_n), constant_values=jnp.int32(B + 7))

    unit_rows = jnp.concatenate([jnp.zeros((CHUNK, 16), F32),
                                 jnp.ones((CHUNK, 16), F32)])

    cnt = _degree_count(dst_st, unit_rows)
    m1, m2 = _onehot(jnp.stack([bp1, bp2]))

    ys = _y0(xs, W1, cnt)
    pools = []
    att = [(a1_w1, a1_b1, a1_w2, a1_b2), (a2_w1, a2_b1, a2_w2, a2_b2),
           (a3_w1, a3_b1, a3_w2, a3_b2)]
    biases = [b1, b2, b3]
    nexts = [W2, W3, None]
    zero_rows = jnp.zeros((CHUNK, 128), F32)
    for lvl in range(3):
        feat = biases[lvl].shape[0]
        if lvl == 0:
            agg = _edge_aggregate(ys, src_st, dst_st, zero_rows, 128)
        else:
            agg = _edge_aggregate_narrow(ys, src_st, dst_st, feat)
        w1a, b1a, w2a, b2a = att[lvl]
        bl = biases[lvl].reshape(1, feat)
        res = _hy(agg, ys, cnt, bl, nexts[lvl])
        if nexts[lvl] is not None:
            ys = res[1]
        pools.append(_pool(res[0], w1a, b1a.reshape(1, -1), w2a,
                           b2a.reshape(1, -1), m1, m2))

    tn = [(tn1_W.reshape(128 * 128, 64), tn1_Wb.T, tn1_bias.reshape(1, -1)),
          (tn2_W.reshape(64 * 64, 32), tn2_Wb.T, tn2_bias.reshape(1, -1)),
          (tn3_W.reshape(32 * 32, 16), tn3_Wb.T, tn3_bias.reshape(1, -1))]
    pool_args = [pools[0][0], pools[0][1], pools[1][0], pools[1][1],
                 pools[2][0], pools[2][1]]
    return _head(pool_args, tn, se_w1, se_b1, se_w2, se_b2, fc_w, fc_b)
